# Initial kernel scaffold; baseline (speedup 1.0000x reference)
import functools

import jax
import jax.numpy as jnp
from jax import lax
from jax.experimental import pallas as pl
from jax.experimental.pallas import tpu as pltpu
from jax.experimental.pallas import tpu_sc as plsc

NC = 2
NS = 16
NW = NC * NS
IDXW = 128
CH_ROWS = 256
NBUF = 4
G = CH_ROWS // IDXW


@functools.lru_cache(maxsize=None)
def _make_gather(B, V, D, interpret=False):
    assert B % (NW * CH_ROWS) == 0
    b_per_w = B // NW
    n_idx_rows = b_per_w // IDXW
    nch = b_per_w // CH_ROWS
    assert nch % NBUF == 0 and nch >= 2 * NBUF
    nq = nch // NBUF - 1

    mesh = plsc.VectorSubcoreMesh(
        core_axis_name="c", subcore_axis_name="s",
        num_cores=NC, num_subcores=NS)

    @functools.partial(
        pl.kernel,
        out_type=jax.ShapeDtypeStruct((B, D), jnp.float32),
        mesh=mesh,
        interpret=interpret,
        scratch_types=[
            pltpu.VMEM((n_idx_rows, IDXW), jnp.int32),
            [pltpu.VMEM((CH_ROWS, D), jnp.float32) for _ in range(NBUF)],
            [pltpu.SemaphoreType.DMA for _ in range(NBUF)],
        ],
    )
    def gather_kernel(table_hbm, idx_hbm, out_hbm, idx_v, bufs, gsems):
        wid = lax.axis_index("s") * NC + lax.axis_index("c")
        row_base = wid * b_per_w

        pltpu.sync_copy(idx_hbm.at[pl.ds(wid * n_idx_rows, n_idx_rows)], idx_v)

        def fire_slot(chunk, slot):
            for g in range(G):
                pltpu.async_copy(
                    table_hbm.at[idx_v.at[chunk * G + g]],
                    bufs[slot].at[pl.ds(g * IDXW, IDXW)],
                    gsems[slot])

        def wait_slot(slot):
            pltpu.make_async_copy(
                table_hbm.at[pl.ds(0, CH_ROWS)], bufs[slot], gsems[slot]
            ).wait()

        def store(chunk, slot):
            pltpu.sync_copy(
                bufs[slot],
                out_hbm.at[pl.ds(row_base + chunk * CH_ROWS, CH_ROWS)])

        for c in range(NBUF - 1):
            fire_slot(c, c)

        @pl.loop(0, nq)
        def _(q):
            i0 = q * NBUF
            for r in range(NBUF):
                wait_slot(r)
                fire_slot(i0 + r + NBUF - 1, (r + NBUF - 1) % NBUF)
                store(i0 + r, r)

        i0 = nq * NBUF
        fire_slot(i0 + NBUF - 1, NBUF - 1)
        for r in range(NBUF):
            wait_slot(r)
            store(i0 + r, r)

    return gather_kernel


def kernel(token_ids, weight):
    B, H = token_ids.shape
    V, D = weight.shape
    flat = token_ids.reshape(-1).astype(jnp.int32)
    idx2d = flat.reshape(-1, IDXW)
    out = _make_gather(flat.shape[0], V, D)(weight, idx2d)
    return out.reshape(B, H, D)


# speedup vs baseline: 1.8738x; 1.8738x over previous
import functools

import jax
import jax.numpy as jnp
from jax import lax
from jax.experimental import pallas as pl
from jax.experimental.pallas import tpu as pltpu
from jax.experimental.pallas import tpu_sc as plsc

NC = 2
NS = 16
NW = NC * NS
IDXW = 128
CH_ROWS = 256
NBUF = 4
G = CH_ROWS // IDXW


@functools.lru_cache(maxsize=None)
def _make_gather(B, V, D, interpret=False):
    assert B % (NW * CH_ROWS) == 0
    b_per_w = B // NW
    n_idx_rows = b_per_w // IDXW
    nch = b_per_w // CH_ROWS
    assert nch % NBUF == 0 and nch >= 2 * NBUF
    nq = nch // NBUF - 1

    mesh = plsc.VectorSubcoreMesh(
        core_axis_name="c", subcore_axis_name="s",
        num_cores=NC, num_subcores=NS)

    @functools.partial(
        pl.kernel,
        out_type=jax.ShapeDtypeStruct((B, D), jnp.float32),
        mesh=mesh,
        interpret=interpret,
        compiler_params=pltpu.CompilerParams(use_tc_tiling_on_sc=False),
        scratch_types=[
            pltpu.VMEM((n_idx_rows, IDXW), jnp.int32),
            [pltpu.VMEM((CH_ROWS, D), jnp.float32) for _ in range(NBUF)],
            [pltpu.SemaphoreType.DMA for _ in range(NBUF)],
        ],
    )
    def gather_kernel(table_hbm, idx_hbm, out_hbm, idx_v, bufs, gsems):
        wid = lax.axis_index("s") * NC + lax.axis_index("c")
        row_base = wid * b_per_w

        pltpu.sync_copy(idx_hbm.at[pl.ds(wid * n_idx_rows, n_idx_rows)], idx_v)

        def fire_slot(chunk, slot):
            for g in range(G):
                pltpu.async_copy(
                    table_hbm.at[idx_v.at[chunk * G + g]],
                    bufs[slot].at[pl.ds(g * IDXW, IDXW)],
                    gsems[slot])

        def wait_slot(slot):
            pltpu.make_async_copy(
                table_hbm.at[pl.ds(0, CH_ROWS)], bufs[slot], gsems[slot]
            ).wait()

        def store(chunk, slot):
            pltpu.sync_copy(
                bufs[slot],
                out_hbm.at[pl.ds(row_base + chunk * CH_ROWS, CH_ROWS)])

        for c in range(NBUF - 1):
            fire_slot(c, c)

        @pl.loop(0, nq)
        def _(q):
            i0 = q * NBUF
            for r in range(NBUF):
                wait_slot(r)
                fire_slot(i0 + r + NBUF - 1, (r + NBUF - 1) % NBUF)
                store(i0 + r, r)

        i0 = nq * NBUF
        fire_slot(i0 + NBUF - 1, NBUF - 1)
        for r in range(NBUF):
            wait_slot(r)
            store(i0 + r, r)

    return gather_kernel


def kernel(token_ids, weight):
    B, H = token_ids.shape
    V, D = weight.shape
    flat = token_ids.reshape(-1).astype(jnp.int32)
    idx2d = flat.reshape(-1, IDXW)
    out = _make_gather(flat.shape[0], V, D)(weight, idx2d)
    return out.reshape(B, H, D)
